# Initial kernel scaffold; baseline (speedup 1.0000x reference)
import jax
import jax.numpy as jnp
from jax.experimental import pallas as pl


def kernel(x, table):
    raise NotImplementedError("write your pallas kernel here")


# speedup vs baseline: 3.3277x; 3.3277x over previous
import functools

import jax
import jax.numpy as jnp
from jax import lax
from jax.experimental import pallas as pl
from jax.experimental.pallas import tpu as pltpu
from jax.experimental.pallas import tpu_sc as plsc

D = 64
ROWW = 128
GRP = 512
SUB = GRP // ROWW
NC, NS = 2, 16
NW = NC * NS
L = 16


@functools.lru_cache(maxsize=None)
def _emb_kernel(B):
    assert B % (NW * GRP) == 0
    b_per_w = B // NW
    n_grp = b_per_w // GRP
    rows_per_w = b_per_w // ROWW
    mesh = plsc.VectorSubcoreMesh(core_axis_name="c", subcore_axis_name="s")

    @functools.partial(
        pl.kernel,
        mesh=mesh,
        out_type=(
            jax.ShapeDtypeStruct((B, D), jnp.float32),
            jax.ShapeDtypeStruct((B // ROWW, ROWW), jnp.float32),
        ),
        scratch_types=[
            pltpu.VMEM((SUB, ROWW), jnp.int32),
            pltpu.VMEM((GRP, D), jnp.float32),
            pltpu.VMEM((SUB, ROWW), jnp.float32),
            pltpu.SemaphoreType.DMA,
        ],
        compiler_params=pltpu.CompilerParams(use_tc_tiling_on_sc=False),
    )
    def k(idx_hbm, table_hbm, emb_hbm, mask_hbm, idx_v, rows_v, mask_v, sem):
        wid = lax.axis_index("s") * NC + lax.axis_index("c")
        row0 = wid * rows_per_w

        def body(g, carry):
            r = row0 + g * SUB
            pltpu.sync_copy(idx_hbm.at[pl.ds(r, SUB)], idx_v)
            for j in range(SUB):
                pltpu.async_copy(
                    table_hbm.at[idx_v.at[j]],
                    rows_v.at[pl.ds(j * ROWW, ROWW)],
                    sem,
                ).wait()
                for t in range(ROWW // L):
                    iv = idx_v[j, pl.ds(t * L, L)]
                    mask_v[j, pl.ds(t * L, L)] = jnp.where(
                        iv != 0, jnp.float32(1.0), jnp.float32(0.0))
            pltpu.sync_copy(rows_v, emb_hbm.at[pl.ds(r * ROWW, GRP)])
            pltpu.sync_copy(mask_v, mask_hbm.at[pl.ds(r, SUB)])
            return carry

        lax.fori_loop(0, n_grp, body, 0)

    return k


def kernel(x, table):
    B = x.size
    idx = x.reshape(B // ROWW, ROWW).astype(jnp.int32)
    emb, mask = _emb_kernel(B)(idx, table.astype(jnp.float32))
    return emb.reshape(*x.shape, D), mask.reshape(x.shape)


# speedup vs baseline: 3.4920x; 1.0494x over previous
import functools

import jax
import jax.numpy as jnp
from jax import lax
from jax.experimental import pallas as pl
from jax.experimental.pallas import tpu as pltpu
from jax.experimental.pallas import tpu_sc as plsc

D = 64
ROWW = 128
GRP = 512
SUB = GRP // ROWW
NC, NS = 2, 16
NW = NC * NS
L = 16


@functools.lru_cache(maxsize=None)
def _emb_kernel(B):
    assert B % (NW * GRP * 2) == 0
    b_per_w = B // NW
    n_grp = b_per_w // GRP
    rows_per_w = b_per_w // ROWW
    mesh = plsc.VectorSubcoreMesh(core_axis_name="c", subcore_axis_name="s")

    @functools.partial(
        pl.kernel,
        mesh=mesh,
        out_type=(
            jax.ShapeDtypeStruct((B, D), jnp.float32),
            jax.ShapeDtypeStruct((B // ROWW, ROWW), jnp.float32),
        ),
        scratch_types=[
            pltpu.VMEM((2, SUB, ROWW), jnp.int32),
            pltpu.VMEM((2, GRP, D), jnp.float32),
            pltpu.VMEM((2, SUB, ROWW), jnp.float32),
            pltpu.SemaphoreType.DMA((2,)),
            pltpu.SemaphoreType.DMA((2,)),
            pltpu.SemaphoreType.DMA((2,)),
            pltpu.SemaphoreType.DMA((2,)),
        ],
        compiler_params=pltpu.CompilerParams(use_tc_tiling_on_sc=False),
    )
    def k(idx_hbm, table_hbm, emb_hbm, mask_hbm,
          idx_v, rows_v, mask_v, s_idx, s_gat, s_emb, s_msk):
        wid = lax.axis_index("s") * NC + lax.axis_index("c")
        row0 = wid * rows_per_w

        def idx_copy(g, b):
            return pltpu.make_async_copy(
                idx_hbm.at[pl.ds(row0 + g * SUB, SUB)],
                idx_v.at[b], s_idx.at[b])

        def emb_copy(g, b):
            return pltpu.make_async_copy(
                rows_v.at[b], emb_hbm.at[pl.ds((row0 + g * SUB) * ROWW, GRP)],
                s_emb.at[b])

        def mask_copy(g, b):
            return pltpu.make_async_copy(
                mask_v.at[b], mask_hbm.at[pl.ds(row0 + g * SUB, SUB)],
                s_msk.at[b])

        idx_copy(0, 0).start()

        def run_group(g, b):
            idx_copy(g, b).wait()
            if b == 0:
                idx_copy(g + 1, 1).start()
            else:
                @pl.when(g + 1 < n_grp)
                def _():
                    idx_copy(g + 1, 0).start()
            @pl.when(g >= 2)
            def _():
                emb_copy(g - 2, b).wait()
                mask_copy(g - 2, b).wait()
            gathers = []
            for j in range(SUB):
                gathers.append(pltpu.async_copy(
                    table_hbm.at[idx_v.at[b, j]],
                    rows_v.at[b, pl.ds(j * ROWW, ROWW)],
                    s_gat.at[b]))
            for j in range(SUB):
                for t in range(ROWW // L):
                    iv = idx_v[b, j, pl.ds(t * L, L)]
                    mask_v[b, j, pl.ds(t * L, L)] = jnp.where(
                        iv != 0, jnp.float32(1.0), jnp.float32(0.0))
            for c in gathers:
                c.wait()
            emb_copy(g, b).start()
            mask_copy(g, b).start()

        def pair(i, carry):
            g0 = i * 2
            run_group(g0, 0)
            run_group(g0 + 1, 1)
            return carry

        lax.fori_loop(0, n_grp // 2, pair, 0)
        for b in range(2):
            emb_copy(n_grp - 2 + b, b).wait()
            mask_copy(n_grp - 2 + b, b).wait()

    return k


def kernel(x, table):
    B = x.size
    idx = x.reshape(B // ROWW, ROWW).astype(jnp.int32)
    emb, mask = _emb_kernel(B)(idx, table.astype(jnp.float32))
    return emb.reshape(*x.shape, D), mask.reshape(x.shape)


# speedup vs baseline: 4.7511x; 1.3606x over previous
import functools

import jax
import jax.numpy as jnp
from jax import lax
from jax.experimental import pallas as pl
from jax.experimental.pallas import tpu as pltpu
from jax.experimental.pallas import tpu_sc as plsc

D = 64
ROWW = 128
GRP = 256
NROW = 1000
SUB = GRP // ROWW
NC, NS = 2, 16
NW = NC * NS
L = 16


@functools.lru_cache(maxsize=None)
def _emb_kernel(B):
    assert B % (NW * GRP * 2) == 0
    b_per_w = B // NW
    n_grp = b_per_w // GRP
    rows_per_w = b_per_w // ROWW
    mesh = plsc.VectorSubcoreMesh(core_axis_name="c", subcore_axis_name="s")

    @functools.partial(
        pl.kernel,
        mesh=mesh,
        out_type=(
            jax.ShapeDtypeStruct((B, D), jnp.float32),
            jax.ShapeDtypeStruct((B // ROWW, ROWW), jnp.float32),
        ),
        scratch_types=[
            pltpu.VMEM_SHARED((NROW, D), jnp.float32),
            pltpu.VMEM((2, SUB, ROWW), jnp.int32),
            pltpu.VMEM((2, GRP, D), jnp.float32),
            pltpu.VMEM((2, SUB, ROWW), jnp.float32),
            pltpu.SemaphoreType.DMA((2,)),
            pltpu.SemaphoreType.DMA((2,)),
            pltpu.SemaphoreType.DMA((2,)),
            pltpu.SemaphoreType.DMA((2,)),
        ],
        compiler_params=pltpu.CompilerParams(use_tc_tiling_on_sc=False),
    )
    def k(idx_hbm, table_hbm, emb_hbm, mask_hbm,
          table_v, idx_v, rows_v, mask_v, s_idx, s_gat, s_emb, s_msk):
        wid = lax.axis_index("s") * NC + lax.axis_index("c")
        row0 = wid * rows_per_w
        @pl.when(lax.axis_index("s") == 0)
        def _():
            pltpu.sync_copy(table_hbm, table_v)
        plsc.subcore_barrier()

        def idx_copy(g, b):
            return pltpu.make_async_copy(
                idx_hbm.at[pl.ds(row0 + g * SUB, SUB)],
                idx_v.at[b], s_idx.at[b])

        def emb_copy(g, b):
            return pltpu.make_async_copy(
                rows_v.at[b], emb_hbm.at[pl.ds((row0 + g * SUB) * ROWW, GRP)],
                s_emb.at[b])

        def mask_copy(g, b):
            return pltpu.make_async_copy(
                mask_v.at[b], mask_hbm.at[pl.ds(row0 + g * SUB, SUB)],
                s_msk.at[b])

        idx_copy(0, 0).start()

        def run_group(g, b):
            idx_copy(g, b).wait()
            if b == 0:
                idx_copy(g + 1, 1).start()
            else:
                @pl.when(g + 1 < n_grp)
                def _():
                    idx_copy(g + 1, 0).start()
            @pl.when(g >= 2)
            def _():
                emb_copy(g - 2, b).wait()
                mask_copy(g - 2, b).wait()
            gathers = []
            for j in range(SUB):
                gathers.append(pltpu.async_copy(
                    table_v.at[idx_v.at[b, j]],
                    rows_v.at[b, pl.ds(j * ROWW, ROWW)],
                    s_gat.at[b]))
            for j in range(SUB):
                for t in range(ROWW // L):
                    iv = idx_v[b, j, pl.ds(t * L, L)]
                    mask_v[b, j, pl.ds(t * L, L)] = jnp.where(
                        iv != 0, jnp.float32(1.0), jnp.float32(0.0))
            for c in gathers:
                c.wait()
            emb_copy(g, b).start()
            mask_copy(g, b).start()

        def pair(i, carry):
            g0 = i * 2
            run_group(g0, 0)
            run_group(g0 + 1, 1)
            return carry

        lax.fori_loop(0, n_grp // 2, pair, 0)
        for b in range(2):
            emb_copy(n_grp - 2 + b, b).wait()
            mask_copy(n_grp - 2 + b, b).wait()

    return k


def kernel(x, table):
    B = x.size
    idx = x.reshape(B // ROWW, ROWW).astype(jnp.int32)
    emb, mask = _emb_kernel(B)(idx, table.astype(jnp.float32))
    return emb.reshape(*x.shape, D), mask.reshape(x.shape)
